# baseline (speedup 1.0000x reference)
import functools

import jax
import jax.numpy as jnp
from jax import lax
from jax.experimental import pallas as pl
from jax.experimental.pallas import tpu as pltpu
from jax.experimental.pallas import tpu_sc as plsc

_B = 16
_N = 512 * 512
_K = int(0.15 * _N)
_HALF = _N // 2
_NROW = 256
_CHROW = 128
_NCHUNK = _NROW // _CHROW


def _bce_bits_body(o_ref, t_ref, bits_ref):
    o = o_ref[...]
    t = t_ref[...]
    loss = jnp.maximum(o, 0.0) - o * t + jnp.log1p(jnp.exp(-jnp.abs(o)))
    bits_ref[...] = lax.bitcast_convert_type(loss, jnp.int32)


def _bce_bits(o3, t3):
    return pl.pallas_call(
        _bce_bits_body,
        grid=(4,),
        in_specs=[
            pl.BlockSpec((8, _NROW, 512), lambda i: (i, 0, 0)),
            pl.BlockSpec((8, _NROW, 512), lambda i: (i, 0, 0)),
        ],
        out_specs=pl.BlockSpec((8, _NROW, 512), lambda i: (i, 0, 0)),
        out_shape=jax.ShapeDtypeStruct((2 * _B, _NROW, 512), jnp.int32),
    )(o3, t3)


_LANES = 16


def _zero_hist(hc, hs, nvec):
    zi = jnp.zeros((_LANES,), jnp.int32)
    zf = jnp.zeros((_LANES,), jnp.float32)

    def body(i, _):
        off = pl.multiple_of(i * _LANES, _LANES)
        hc[pl.ds(off, _LANES)] = zi
        hs[pl.ds(off, _LANES)] = zf
        return 0

    lax.fori_loop(0, nvec, body, 0)


def _merge_partner(mine_c, mine_s, sh_c, sh_s, pbc, pbs, s_id, nvec):
    pltpu.sync_copy(mine_c, sh_c.at[s_id])
    pltpu.sync_copy(mine_s, sh_s.at[s_id])
    plsc.subcore_barrier()
    partner = jnp.bitwise_xor(s_id, 1)
    pltpu.sync_copy(sh_c.at[partner], pbc)
    pltpu.sync_copy(sh_s.at[partner], pbs)

    def body(i, _):
        off = pl.multiple_of(i * _LANES, _LANES)
        mine_c[pl.ds(off, _LANES)] = mine_c[pl.ds(off, _LANES)] + pbc[pl.ds(off, _LANES)]
        mine_s[pl.ds(off, _LANES)] = mine_s[pl.ds(off, _LANES)] + pbs[pl.ds(off, _LANES)]
        return 0

    lax.fori_loop(0, nvec, body, 0)
    plsc.subcore_barrier()


def _find_bin(hc, nvec, klev):
    def body(t, carry):
        s_run, i_star, s_excl = carry
        j = nvec - 1 - t
        v = hc[pl.ds(pl.multiple_of(j * _LANES, _LANES), _LANES)]
        s_new = s_run + jnp.sum(v)
        hit = jnp.logical_and(s_new >= klev, s_run < klev)
        return (s_new,
                jnp.where(hit, j, i_star),
                jnp.where(hit, s_run, s_excl))

    _, i_star, s_excl = lax.fori_loop(
        0, nvec, body, (jnp.int32(0), jnp.int32(0), jnp.int32(0)))

    v = hc[pl.ds(pl.multiple_of(i_star * _LANES, _LANES), _LANES)]
    pref = jnp.cumsum(v)
    tot = jnp.sum(v)
    suff = tot - pref + v
    need = klev - s_excl
    mask = suff >= need
    lane = jnp.sum(mask.astype(jnp.int32)) - 1
    lanes = lax.iota(jnp.int32, _LANES)
    sel = lanes == lane
    suff_at = jnp.sum(jnp.where(sel, suff, 0))
    cnt_at = jnp.sum(jnp.where(sel, v, 0))
    b = i_star * _LANES + lane
    cnt_above = s_excl + suff_at - cnt_at
    return b, cnt_above


def _suffix_sum_f32(hs, nvec, b):
    lanes = lax.iota(jnp.int32, _LANES)

    def body(j, acc):
        vs = hs[pl.ds(pl.multiple_of(j * _LANES, _LANES), _LANES)]
        bins = j * _LANES + lanes
        return acc + jnp.sum(jnp.where(bins > b, vs, 0.0))

    return lax.fori_loop(0, nvec, body, jnp.float32(0.0))


def _level3_value_sum(hc, b3, hi_bits):
    lanes = lax.iota(jnp.int32, _LANES)

    def body(j, acc):
        vc = hc[pl.ds(pl.multiple_of(j * _LANES, _LANES), _LANES)]
        bins = j * _LANES + lanes
        vals = plsc.bitcast(jnp.bitwise_or(bins, hi_bits), jnp.float32)
        return acc + jnp.sum(jnp.where(bins > b3, vals * vc.astype(jnp.float32), 0.0))

    return lax.fori_loop(0, 8, body, jnp.float32(0.0))


def _make_sc_topk():
    mesh = plsc.VectorSubcoreMesh(core_axis_name="c", subcore_axis_name="s")

    @functools.partial(
        pl.kernel,
        out_type=jax.ShapeDtypeStruct((_B, _LANES), jnp.float32),
        mesh=mesh,
        compiler_params=pltpu.CompilerParams(needs_layout_passes=False),
        scratch_types=[
            pltpu.VMEM((_CHROW, 512), jnp.int32),
            pltpu.VMEM((4096,), jnp.int32),
            pltpu.VMEM((4096,), jnp.float32),
            pltpu.VMEM((4096,), jnp.int32),
            pltpu.VMEM((4096,), jnp.float32),
            pltpu.VMEM((_LANES,), jnp.float32),
            pltpu.VMEM_SHARED((16, 4096), jnp.int32),
            pltpu.VMEM_SHARED((16, 4096), jnp.float32),
        ],
    )
    def sc_topk(bits_hbm, out_hbm, dbuf, hc, hs, pbc, pbs, cbuf, sh_c, sh_s):
        c_id = lax.axis_index("c")
        s_id = lax.axis_index("s")
        vr = c_id * 16 + s_id
        ones = jnp.full((_LANES,), 1, jnp.int32)

        def stream(process_vec):
            def chunk_body(ck, _):
                pltpu.sync_copy(
                    bits_hbm.at[vr, pl.ds(ck * _CHROW, _CHROW), :], dbuf)

                def row_body(i, _):
                    for j in range(512 // _LANES):
                        v = dbuf[i, pl.ds(j * _LANES, _LANES)]
                        process_vec(v)
                    return 0

                lax.fori_loop(0, _CHROW, row_body, 0)
                return 0

            lax.fori_loop(0, _NCHUNK, chunk_body, 0)

        _zero_hist(hc, hs, 256)

        def p1(v):
            idx = lax.shift_right_logical(v, 19)
            plsc.addupdate_scatter(hc, [idx], ones)
            plsc.addupdate_scatter(hs, [idx], plsc.bitcast(v, jnp.float32))

        stream(p1)
        _merge_partner(hc, hs, sh_c, sh_s, pbc, pbs, s_id, 256)
        b1, ca1 = _find_bin(hc, 256, jnp.int32(_K))
        sa1 = _suffix_sum_f32(hs, 256, b1)
        k2 = _K - ca1

        _zero_hist(hc, hs, 256)

        def p2(v):
            m = lax.shift_right_logical(v, 19) == b1
            idx = jnp.bitwise_and(lax.shift_right_logical(v, 7), 0xFFF)
            plsc.addupdate_scatter(hc, [idx], ones, mask=m)
            plsc.addupdate_scatter(hs, [idx], plsc.bitcast(v, jnp.float32), mask=m)

        stream(p2)
        _merge_partner(hc, hs, sh_c, sh_s, pbc, pbs, s_id, 256)
        b2, ca2 = _find_bin(hc, 256, k2)
        sa2 = _suffix_sum_f32(hs, 256, b2)
        k3 = k2 - ca2

        _zero_hist(hc, hs, 8)
        hi20 = jnp.bitwise_or(lax.shift_left(b1, 12), b2)

        def p3(v):
            m = lax.shift_right_logical(v, 7) == hi20
            idx = jnp.bitwise_and(v, 0x7F)
            plsc.addupdate_scatter(hc, [idx], ones, mask=m)

        stream(p3)
        _merge_partner(hc, hs, sh_c, sh_s, pbc, pbs, s_id, 8)
        b3, ca3 = _find_bin(hc, 8, k3)
        hi_bits = jnp.bitwise_or(lax.shift_left(b1, 19), lax.shift_left(b2, 7))
        sa3 = _level3_value_sum(hc, b3, hi_bits)

        cnt_gt = ca1 + ca2 + ca3
        vk_bits = jnp.bitwise_or(hi_bits, b3)
        vk = plsc.bitcast(jnp.full((_LANES,), vk_bits, jnp.int32), jnp.float32)
        contrib = sa1 + sa2 + sa3 + (_K - cnt_gt).astype(jnp.float32) * vk
        cbuf[...] = contrib

        @pl.when(jnp.bitwise_and(vr, 1) == 0)
        def _():
            row = lax.shift_right_logical(vr, 1)
            pltpu.sync_copy(cbuf, out_hbm.at[row])

    return sc_topk


_sc_topk_call = _make_sc_topk()


@jax.jit
def _full(output, target):
    o3 = output.reshape(2 * _B, _NROW, 512)
    t3 = target.reshape(2 * _B, _NROW, 512)
    bits = _bce_bits(o3, t3)
    rows = _sc_topk_call(bits)
    return jnp.sum(rows[:, 0]) / jnp.float32(_B * _K)


def kernel(output, target):
    return _full(output, target)


# speedup vs baseline: 2.2097x; 2.2097x over previous
import functools

import jax
import jax.numpy as jnp
from jax import lax
from jax.experimental import pallas as pl
from jax.experimental.pallas import tpu as pltpu
from jax.experimental.pallas import tpu_sc as plsc

_B = 16
_N = 512 * 512
_K = int(0.15 * _N)
_HALF = _N // 2
_NROW = 256
_CHROW = 128
_NCHUNK = _NROW // _CHROW


def _bce_bits_body(o_ref, t_ref, bits_ref):
    o = o_ref[...]
    t = t_ref[...]
    loss = jnp.maximum(o, 0.0) - o * t + jnp.log1p(jnp.exp(-jnp.abs(o)))
    bits_ref[...] = lax.bitcast_convert_type(loss, jnp.int32)


def _bce_bits(o3, t3):
    return pl.pallas_call(
        _bce_bits_body,
        grid=(4,),
        in_specs=[
            pl.BlockSpec((8, _NROW, 512), lambda i: (i, 0, 0)),
            pl.BlockSpec((8, _NROW, 512), lambda i: (i, 0, 0)),
        ],
        out_specs=pl.BlockSpec((8, _NROW, 512), lambda i: (i, 0, 0)),
        out_shape=jax.ShapeDtypeStruct((2 * _B, _NROW, 512), jnp.int32),
    )(o3, t3)


_LANES = 16


def _zero_hist(hc, hs, nvec):
    zi = jnp.zeros((_LANES,), jnp.int32)
    zf = jnp.zeros((_LANES,), jnp.float32)

    @plsc.parallel_loop(0, nvec, unroll=4)
    def _(i):
        off = pl.multiple_of(i * _LANES, _LANES)
        hc[pl.ds(off, _LANES)] = zi
        hs[pl.ds(off, _LANES)] = zf


def _merge_partner(mine_c, mine_s, sh_c, sh_s, pbc, pbs, s_id, nvec):
    pltpu.sync_copy(mine_c, sh_c.at[s_id])
    pltpu.sync_copy(mine_s, sh_s.at[s_id])
    plsc.subcore_barrier()
    partner = jnp.bitwise_xor(s_id, 1)
    pltpu.sync_copy(sh_c.at[partner], pbc)
    pltpu.sync_copy(sh_s.at[partner], pbs)

    @plsc.parallel_loop(0, nvec, unroll=4)
    def _(i):
        off = pl.multiple_of(i * _LANES, _LANES)
        mine_c[pl.ds(off, _LANES)] = mine_c[pl.ds(off, _LANES)] + pbc[pl.ds(off, _LANES)]
        mine_s[pl.ds(off, _LANES)] = mine_s[pl.ds(off, _LANES)] + pbs[pl.ds(off, _LANES)]

    plsc.subcore_barrier()


def _find_bin(hc, nvec, klev):
    def body(t, carry):
        s_run, i_star, s_excl = carry
        j = nvec - 1 - t
        v = hc[pl.ds(pl.multiple_of(j * _LANES, _LANES), _LANES)]
        s_new = s_run + jnp.sum(v)
        hit = jnp.logical_and(s_new >= klev, s_run < klev)
        return (s_new,
                jnp.where(hit, j, i_star),
                jnp.where(hit, s_run, s_excl))

    _, i_star, s_excl = lax.fori_loop(
        0, nvec, body, (jnp.int32(0), jnp.int32(0), jnp.int32(0)))

    v = hc[pl.ds(pl.multiple_of(i_star * _LANES, _LANES), _LANES)]
    pref = jnp.cumsum(v)
    tot = jnp.sum(v)
    suff = tot - pref + v
    need = klev - s_excl
    mask = suff >= need
    lane = jnp.sum(mask.astype(jnp.int32)) - 1
    lanes = lax.iota(jnp.int32, _LANES)
    sel = lanes == lane
    suff_at = jnp.sum(jnp.where(sel, suff, 0))
    cnt_at = jnp.sum(jnp.where(sel, v, 0))
    b = i_star * _LANES + lane
    cnt_above = s_excl + suff_at - cnt_at
    return b, cnt_above


def _suffix_sum_f32(hs, nvec, b):
    lanes = lax.iota(jnp.int32, _LANES)

    def body(j, acc):
        vs = hs[pl.ds(pl.multiple_of(j * _LANES, _LANES), _LANES)]
        bins = j * _LANES + lanes
        return acc + jnp.sum(jnp.where(bins > b, vs, 0.0))

    return lax.fori_loop(0, nvec, body, jnp.float32(0.0))


def _level3_value_sum(hc, b3, hi_bits):
    lanes = lax.iota(jnp.int32, _LANES)

    def body(j, acc):
        vc = hc[pl.ds(pl.multiple_of(j * _LANES, _LANES), _LANES)]
        bins = j * _LANES + lanes
        vals = plsc.bitcast(jnp.bitwise_or(bins, hi_bits), jnp.float32)
        return acc + jnp.sum(jnp.where(bins > b3, vals * vc.astype(jnp.float32), 0.0))

    return lax.fori_loop(0, 8, body, jnp.float32(0.0))


def _make_sc_topk():
    mesh = plsc.VectorSubcoreMesh(core_axis_name="c", subcore_axis_name="s")

    @functools.partial(
        pl.kernel,
        out_type=jax.ShapeDtypeStruct((_B, _LANES), jnp.float32),
        mesh=mesh,
        compiler_params=pltpu.CompilerParams(needs_layout_passes=False),
        scratch_types=[
            pltpu.VMEM((_CHROW, 512), jnp.int32),
            pltpu.VMEM((4096,), jnp.int32),
            pltpu.VMEM((4096,), jnp.float32),
            pltpu.VMEM((4096,), jnp.int32),
            pltpu.VMEM((4096,), jnp.float32),
            pltpu.VMEM((_LANES,), jnp.float32),
            pltpu.VMEM_SHARED((16, 4096), jnp.int32),
            pltpu.VMEM_SHARED((16, 4096), jnp.float32),
        ],
    )
    def sc_topk(bits_hbm, out_hbm, dbuf, hc, hs, pbc, pbs, cbuf, sh_c, sh_s):
        c_id = lax.axis_index("c")
        s_id = lax.axis_index("s")
        vr = c_id * 16 + s_id
        ones = jnp.full((_LANES,), 1, jnp.int32)

        def stream(process_vec):
            def chunk_body(ck, _):
                pltpu.sync_copy(
                    bits_hbm.at[vr, pl.ds(ck * _CHROW, _CHROW), :], dbuf)

                @plsc.parallel_loop(0, _CHROW)
                def _(i):
                    for j in range(512 // _LANES):
                        v = dbuf[i, pl.ds(j * _LANES, _LANES)]
                        process_vec(v)

                return 0

            lax.fori_loop(0, _NCHUNK, chunk_body, 0)

        _zero_hist(hc, hs, 256)

        def p1(v):
            idx = lax.shift_right_logical(v, 19)
            plsc.addupdate_scatter(hc, [idx], ones)
            plsc.addupdate_scatter(hs, [idx], plsc.bitcast(v, jnp.float32))

        stream(p1)
        _merge_partner(hc, hs, sh_c, sh_s, pbc, pbs, s_id, 256)
        b1, ca1 = _find_bin(hc, 256, jnp.int32(_K))
        sa1 = _suffix_sum_f32(hs, 256, b1)
        k2 = _K - ca1

        _zero_hist(hc, hs, 256)

        def p2(v):
            m = lax.shift_right_logical(v, 19) == b1
            idx = jnp.bitwise_and(lax.shift_right_logical(v, 7), 0xFFF)
            plsc.addupdate_scatter(hc, [idx], ones, mask=m)
            plsc.addupdate_scatter(hs, [idx], plsc.bitcast(v, jnp.float32), mask=m)

        stream(p2)
        _merge_partner(hc, hs, sh_c, sh_s, pbc, pbs, s_id, 256)
        b2, ca2 = _find_bin(hc, 256, k2)
        sa2 = _suffix_sum_f32(hs, 256, b2)
        k3 = k2 - ca2

        _zero_hist(hc, hs, 8)
        hi20 = jnp.bitwise_or(lax.shift_left(b1, 12), b2)

        def p3(v):
            m = lax.shift_right_logical(v, 7) == hi20
            idx = jnp.bitwise_and(v, 0x7F)
            plsc.addupdate_scatter(hc, [idx], ones, mask=m)

        stream(p3)
        _merge_partner(hc, hs, sh_c, sh_s, pbc, pbs, s_id, 8)
        b3, ca3 = _find_bin(hc, 8, k3)
        hi_bits = jnp.bitwise_or(lax.shift_left(b1, 19), lax.shift_left(b2, 7))
        sa3 = _level3_value_sum(hc, b3, hi_bits)

        cnt_gt = ca1 + ca2 + ca3
        vk_bits = jnp.bitwise_or(hi_bits, b3)
        vk = plsc.bitcast(jnp.full((_LANES,), vk_bits, jnp.int32), jnp.float32)
        contrib = sa1 + sa2 + sa3 + (_K - cnt_gt).astype(jnp.float32) * vk
        cbuf[...] = contrib

        @pl.when(jnp.bitwise_and(vr, 1) == 0)
        def _():
            row = lax.shift_right_logical(vr, 1)
            pltpu.sync_copy(cbuf, out_hbm.at[row])

    return sc_topk


_sc_topk_call = _make_sc_topk()


@jax.jit
def _full(output, target):
    o3 = output.reshape(2 * _B, _NROW, 512)
    t3 = target.reshape(2 * _B, _NROW, 512)
    bits = _bce_bits(o3, t3)
    rows = _sc_topk_call(bits)
    return jnp.sum(rows[:, 0]) / jnp.float32(_B * _K)


def kernel(output, target):
    return _full(output, target)


# speedup vs baseline: 2.7416x; 1.2407x over previous
import functools

import jax
import jax.numpy as jnp
from jax import lax
from jax.experimental import pallas as pl
from jax.experimental.pallas import tpu as pltpu
from jax.experimental.pallas import tpu_sc as plsc

_B = 16
_N = 512 * 512
_K = int(0.15 * _N)
_NROW = 256
_CHROW = 64
_NCHUNK = _NROW // _CHROW
_LANES = 16


def _bce_bits_body(o_ref, t_ref, bits_ref):
    o = o_ref[...]
    t = t_ref[...]
    loss = jnp.maximum(o, 0.0) - o * t + jnp.log1p(jnp.exp(-jnp.abs(o)))
    bits_ref[...] = lax.bitcast_convert_type(loss, jnp.int32)


def _bce_bits(o3, t3):
    return pl.pallas_call(
        _bce_bits_body,
        grid=(4,),
        in_specs=[
            pl.BlockSpec((8, _NROW, 512), lambda i: (i, 0, 0)),
            pl.BlockSpec((8, _NROW, 512), lambda i: (i, 0, 0)),
        ],
        out_specs=pl.BlockSpec((8, _NROW, 512), lambda i: (i, 0, 0)),
        out_shape=jax.ShapeDtypeStruct((2 * _B, _NROW, 512), jnp.int32),
    )(o3, t3)


def _zero_hist(hc, nvec):
    zi = jnp.zeros((_LANES,), jnp.int32)

    @plsc.parallel_loop(0, nvec, unroll=4)
    def _(i):
        hc[pl.ds(pl.multiple_of(i * _LANES, _LANES), _LANES)] = zi


def _merge_partner(mine_c, sh_c, pbc, s_id, nvec):
    pltpu.sync_copy(mine_c, sh_c.at[s_id])
    plsc.subcore_barrier()
    partner = jnp.bitwise_xor(s_id, 1)
    pltpu.sync_copy(sh_c.at[partner], pbc)

    @plsc.parallel_loop(0, nvec, unroll=4)
    def _(i):
        off = pl.multiple_of(i * _LANES, _LANES)
        mine_c[pl.ds(off, _LANES)] = mine_c[pl.ds(off, _LANES)] + pbc[pl.ds(off, _LANES)]

    plsc.subcore_barrier()


def _find_bin(hc, nvec, klev):
    def body(t, carry):
        s_run, i_star, s_excl = carry
        j = nvec - 1 - t
        v = hc[pl.ds(pl.multiple_of(j * _LANES, _LANES), _LANES)]
        s_new = s_run + jnp.sum(v)
        hit = jnp.logical_and(s_new >= klev, s_run < klev)
        return (s_new,
                jnp.where(hit, j, i_star),
                jnp.where(hit, s_run, s_excl))

    _, i_star, s_excl = lax.fori_loop(
        0, nvec, body, (jnp.int32(0), jnp.int32(0), jnp.int32(0)))

    v = hc[pl.ds(pl.multiple_of(i_star * _LANES, _LANES), _LANES)]
    pref = jnp.cumsum(v)
    tot = jnp.sum(v)
    suff = tot - pref + v
    need = klev - s_excl
    mask = suff >= need
    lane = jnp.sum(mask.astype(jnp.int32)) - 1
    lanes = lax.iota(jnp.int32, _LANES)
    sel = lanes == lane
    suff_at = jnp.sum(jnp.where(sel, suff, 0))
    cnt_at = jnp.sum(jnp.where(sel, v, 0))
    b = i_star * _LANES + lane
    cnt_above = s_excl + suff_at - cnt_at
    return b, cnt_above


def _level3_value_sum(hc, b3, hi_bits):
    lanes = lax.iota(jnp.int32, _LANES)

    def body(j, acc):
        vc = hc[pl.ds(pl.multiple_of(j * _LANES, _LANES), _LANES)]
        bins = j * _LANES + lanes
        vals = plsc.bitcast(jnp.bitwise_or(bins, hi_bits), jnp.float32)
        return acc + jnp.sum(jnp.where(bins > b3, vals * vc.astype(jnp.float32), 0.0))

    return lax.fori_loop(0, 8, body, jnp.float32(0.0))


def _make_sc_topk():
    mesh = plsc.VectorSubcoreMesh(core_axis_name="c", subcore_axis_name="s")

    @functools.partial(
        pl.kernel,
        out_type=jax.ShapeDtypeStruct((_B, _LANES), jnp.float32),
        mesh=mesh,
        compiler_params=pltpu.CompilerParams(needs_layout_passes=False),
        scratch_types=[
            pltpu.VMEM((_CHROW, 512), jnp.int32),
            pltpu.VMEM((_CHROW, 512), jnp.int32),
            pltpu.VMEM((4096,), jnp.int32),
            pltpu.VMEM((4096,), jnp.int32),
            pltpu.VMEM((_LANES,), jnp.float32),
            pltpu.VMEM((_LANES,), jnp.float32),
            pltpu.VMEM((_LANES,), jnp.float32),
            pltpu.VMEM_SHARED((16, 4096), jnp.int32),
            pltpu.VMEM_SHARED((16, _LANES), jnp.float32),
            pltpu.SemaphoreType.DMA,
            pltpu.SemaphoreType.DMA,
        ],
    )
    def sc_topk(bits_hbm, out_hbm, dbuf0, dbuf1, hc, pbc, sacc, pacc, cbuf,
                sh_c, sh_f, sem0, sem1):
        c_id = lax.axis_index("c")
        s_id = lax.axis_index("s")
        vr = c_id * 16 + s_id
        ones = jnp.full((_LANES,), 1, jnp.int32)
        bufs = (dbuf0, dbuf1)
        sems = (sem0, sem1)

        npair = _NCHUNK // 2

        def issue(ck, parity):
            pltpu.async_copy(
                bits_hbm.at[vr, pl.ds(ck * _CHROW, _CHROW), :],
                bufs[parity], sems[parity])

        def wait(parity):
            pltpu.make_async_copy(
                bits_hbm.at[vr, pl.ds(0, _CHROW), :],
                bufs[parity], sems[parity]).wait()

        def stream(process_row):
            issue(0, 0)

            def pair_body(t, _):
                ck0 = t * 2
                issue(ck0 + 1, 1)
                wait(0)

                @plsc.parallel_loop(0, _CHROW)
                def _(i):
                    process_row(bufs[0], i)

                @pl.when(t + 1 < npair)
                def _():
                    issue(ck0 + 2, 0)

                wait(1)

                @plsc.parallel_loop(0, _CHROW)
                def _(i):
                    process_row(bufs[1], i)

                return 0

            lax.fori_loop(0, npair, pair_body, 0)

        _zero_hist(hc, 256)
        sacc[...] = jnp.zeros((_LANES,), jnp.float32)

        def p1_row(buf, i):
            for j in range(512 // _LANES):
                v = buf[i, pl.ds(j * _LANES, _LANES)]
                idx = lax.shift_right_logical(v, 19)
                plsc.addupdate_scatter(hc, [idx], ones)

        stream(p1_row)
        _merge_partner(hc, sh_c, pbc, s_id, 256)
        b1, ca1 = _find_bin(hc, 256, jnp.int32(_K))
        k2 = _K - ca1

        _zero_hist(hc, 256)

        def p2_row(buf, i):
            for j in range(512 // _LANES):
                v = buf[i, pl.ds(j * _LANES, _LANES)]
                w = lax.shift_right_logical(v, 7)
                m = lax.shift_right_logical(w, 12) == b1
                idx = jnp.bitwise_and(w, 0xFFF)
                plsc.addupdate_scatter(hc, [idx], ones, mask=m)

        stream(p2_row)
        _merge_partner(hc, sh_c, pbc, s_id, 256)
        b2, ca2 = _find_bin(hc, 256, k2)
        k3 = k2 - ca2

        _zero_hist(hc, 8)
        hi20 = jnp.bitwise_or(lax.shift_left(b1, 12), b2)

        def p3_row(buf, i):
            acc = jnp.zeros((_LANES,), jnp.float32)
            for j in range(512 // _LANES):
                v = buf[i, pl.ds(j * _LANES, _LANES)]
                w = lax.shift_right_logical(v, 7)
                meq = w == hi20
                idx = jnp.bitwise_and(v, 0x7F)
                plsc.addupdate_scatter(hc, [idx], ones, mask=meq)
                val = plsc.bitcast(v, jnp.float32)
                acc = acc + jnp.where(w > hi20, val, 0.0)
            plsc.addupdate_scatter(sacc, [lax.iota(jnp.int32, _LANES)], acc)

        stream(p3_row)
        pltpu.sync_copy(sacc, sh_f.at[s_id])
        _merge_partner(hc, sh_c, pbc, s_id, 8)
        pltpu.sync_copy(sh_f.at[jnp.bitwise_xor(s_id, 1)], pacc)
        b3, ca3 = _find_bin(hc, 8, k3)
        hi_bits = jnp.bitwise_or(lax.shift_left(b1, 19), lax.shift_left(b2, 7))
        sa3 = _level3_value_sum(hc, b3, hi_bits)
        sum12 = jnp.sum(sacc[...]) + jnp.sum(pacc[...])

        cnt_gt = ca1 + ca2 + ca3
        vk_bits = jnp.bitwise_or(hi_bits, b3)
        vk = plsc.bitcast(jnp.full((_LANES,), vk_bits, jnp.int32), jnp.float32)
        contrib = sum12 + sa3 + (_K - cnt_gt).astype(jnp.float32) * vk
        cbuf[...] = contrib

        @pl.when(jnp.bitwise_and(vr, 1) == 0)
        def _():
            row = lax.shift_right_logical(vr, 1)
            pltpu.sync_copy(cbuf, out_hbm.at[row])

    return sc_topk


_sc_topk_call = _make_sc_topk()


@jax.jit
def _full(output, target):
    o3 = output.reshape(2 * _B, _NROW, 512)
    t3 = target.reshape(2 * _B, _NROW, 512)
    bits = _bce_bits(o3, t3)
    rows = _sc_topk_call(bits)
    return jnp.sum(rows[:, 0]) / jnp.float32(_B * _K)


def kernel(output, target):
    return _full(output, target)


# speedup vs baseline: 2.7520x; 1.0038x over previous
import functools

import jax
import jax.numpy as jnp
from jax import lax
from jax.experimental import pallas as pl
from jax.experimental.pallas import tpu as pltpu
from jax.experimental.pallas import tpu_sc as plsc

_B = 16
_N = 512 * 512
_K = int(0.15 * _N)
_NROW = 256
_CHROW = 64
_NCHUNK = _NROW // _CHROW
_LANES = 16


def _bce_bits_body(o_ref, t_ref, bits_ref):
    o = o_ref[...]
    t = t_ref[...]
    loss = jnp.maximum(o, 0.0) - o * t + jnp.log1p(jnp.exp(-jnp.abs(o)))
    bits_ref[...] = lax.bitcast_convert_type(loss, jnp.int32)


def _bce_bits(o3, t3):
    return pl.pallas_call(
        _bce_bits_body,
        grid=(4,),
        in_specs=[
            pl.BlockSpec((8, _NROW, 512), lambda i: (i, 0, 0)),
            pl.BlockSpec((8, _NROW, 512), lambda i: (i, 0, 0)),
        ],
        out_specs=pl.BlockSpec((8, _NROW, 512), lambda i: (i, 0, 0)),
        out_shape=jax.ShapeDtypeStruct((2 * _B, _NROW, 512), jnp.int32),
    )(o3, t3)


def _zero_hist(hc, nvec):
    zi = jnp.zeros((_LANES,), jnp.int32)

    @plsc.parallel_loop(0, nvec, unroll=4)
    def _(i):
        hc[pl.ds(pl.multiple_of(i * _LANES, _LANES), _LANES)] = zi


def _merge_partner(mine_c, sh_c, pbc, s_id, nvec):
    pltpu.sync_copy(mine_c, sh_c.at[s_id])
    plsc.subcore_barrier()
    partner = jnp.bitwise_xor(s_id, 1)
    pltpu.sync_copy(sh_c.at[partner], pbc)

    @plsc.parallel_loop(0, nvec, unroll=4)
    def _(i):
        off = pl.multiple_of(i * _LANES, _LANES)
        mine_c[pl.ds(off, _LANES)] = mine_c[pl.ds(off, _LANES)] + pbc[pl.ds(off, _LANES)]

    plsc.subcore_barrier()


def _find_bin(hc, nvec, klev):
    def body(t, carry):
        s_run, i_star, s_excl = carry
        j = nvec - 1 - t
        v = hc[pl.ds(pl.multiple_of(j * _LANES, _LANES), _LANES)]
        s_new = s_run + jnp.sum(v)
        hit = jnp.logical_and(s_new >= klev, s_run < klev)
        return (s_new,
                jnp.where(hit, j, i_star),
                jnp.where(hit, s_run, s_excl))

    _, i_star, s_excl = lax.fori_loop(
        0, nvec, body, (jnp.int32(0), jnp.int32(0), jnp.int32(0)))

    v = hc[pl.ds(pl.multiple_of(i_star * _LANES, _LANES), _LANES)]
    pref = jnp.cumsum(v)
    tot = jnp.sum(v)
    suff = tot - pref + v
    need = klev - s_excl
    mask = suff >= need
    lane = jnp.sum(mask.astype(jnp.int32)) - 1
    lanes = lax.iota(jnp.int32, _LANES)
    sel = lanes == lane
    suff_at = jnp.sum(jnp.where(sel, suff, 0))
    cnt_at = jnp.sum(jnp.where(sel, v, 0))
    b = i_star * _LANES + lane
    cnt_above = s_excl + suff_at - cnt_at
    return b, cnt_above


def _level3_value_sum(hc, b3, hi_bits):
    lanes = lax.iota(jnp.int32, _LANES)

    def body(j, acc):
        vc = hc[pl.ds(pl.multiple_of(j * _LANES, _LANES), _LANES)]
        bins = j * _LANES + lanes
        vals = plsc.bitcast(jnp.bitwise_or(bins, hi_bits), jnp.float32)
        return acc + jnp.sum(jnp.where(bins > b3, vals * vc.astype(jnp.float32), 0.0))

    return lax.fori_loop(0, 8, body, jnp.float32(0.0))


def _make_sc_topk():
    mesh = plsc.VectorSubcoreMesh(core_axis_name="c", subcore_axis_name="s")

    @functools.partial(
        pl.kernel,
        out_type=jax.ShapeDtypeStruct((_B, _LANES), jnp.float32),
        mesh=mesh,
        compiler_params=pltpu.CompilerParams(needs_layout_passes=False),
        scratch_types=[
            pltpu.VMEM((_CHROW, 512), jnp.int32),
            pltpu.VMEM((_CHROW, 512), jnp.int32),
            pltpu.VMEM((4096,), jnp.int32),
            pltpu.VMEM((4096,), jnp.int32),
            pltpu.VMEM((_LANES,), jnp.float32),
            pltpu.VMEM((_LANES,), jnp.float32),
            pltpu.VMEM_SHARED((16, 4096), jnp.int32),
            pltpu.SemaphoreType.DMA,
            pltpu.SemaphoreType.DMA,
        ],
    )
    def sc_topk(bits_hbm, out_hbm, dbuf0, dbuf1, hc, pbc, sacc, cbuf,
                sh_c, sem0, sem1):
        c_id = lax.axis_index("c")
        s_id = lax.axis_index("s")
        vr = c_id * 16 + s_id
        ones = jnp.full((_LANES,), 1, jnp.int32)
        bufs = (dbuf0, dbuf1)
        sems = (sem0, sem1)

        npair = _NCHUNK // 2

        def issue(ck, parity):
            pltpu.async_copy(
                bits_hbm.at[vr, pl.ds(ck * _CHROW, _CHROW), :],
                bufs[parity], sems[parity])

        def wait(parity):
            pltpu.make_async_copy(
                bits_hbm.at[vr, pl.ds(0, _CHROW), :],
                bufs[parity], sems[parity]).wait()

        def stream(process_row):
            issue(0, 0)

            def pair_body(t, _):
                ck0 = t * 2
                issue(ck0 + 1, 1)
                wait(0)

                @plsc.parallel_loop(0, _CHROW)
                def _(i):
                    process_row(bufs[0], i)

                @pl.when(t + 1 < npair)
                def _():
                    issue(ck0 + 2, 0)

                wait(1)

                @plsc.parallel_loop(0, _CHROW)
                def _(i):
                    process_row(bufs[1], i)

                return 0

            lax.fori_loop(0, npair, pair_body, 0)

        _zero_hist(hc, 256)
        sacc[...] = jnp.zeros((_LANES,), jnp.float32)

        def p1_row(buf, i):
            for j in range(512 // _LANES):
                v = buf[i, pl.ds(j * _LANES, _LANES)]
                idx = lax.shift_right_logical(v, 19)
                plsc.addupdate_scatter(hc, [idx], ones)

        stream(p1_row)
        _merge_partner(hc, sh_c, pbc, s_id, 256)
        b1, ca1 = _find_bin(hc, 256, jnp.int32(_K))
        k2 = _K - ca1

        _zero_hist(hc, 256)

        def p2_row(buf, i):
            for j in range(512 // _LANES):
                v = buf[i, pl.ds(j * _LANES, _LANES)]
                w = lax.shift_right_logical(v, 7)
                m = lax.shift_right_logical(w, 12) == b1
                idx = jnp.bitwise_and(w, 0xFFF)
                plsc.addupdate_scatter(hc, [idx], ones, mask=m)

        stream(p2_row)
        _merge_partner(hc, sh_c, pbc, s_id, 256)
        b2, ca2 = _find_bin(hc, 256, k2)
        k3 = k2 - ca2

        _zero_hist(hc, 8)
        hi20 = jnp.bitwise_or(lax.shift_left(b1, 12), b2)

        def p3_row(buf, i):
            acc = jnp.zeros((_LANES,), jnp.float32)
            for j in range(512 // _LANES):
                v = buf[i, pl.ds(j * _LANES, _LANES)]
                w = lax.shift_right_logical(v, 7)
                meq = w == hi20
                idx = jnp.bitwise_and(v, 0x7F)
                plsc.addupdate_scatter(hc, [idx], ones, mask=meq)
                val = plsc.bitcast(v, jnp.float32)
                acc = acc + jnp.where(w > hi20, val, 0.0)
            plsc.addupdate_scatter(sacc, [lax.iota(jnp.int32, _LANES)], acc)

        stream(p3_row)
        hc[pl.ds(128, _LANES)] = plsc.bitcast(sacc[...], jnp.int32)
        _merge_partner(hc, sh_c, pbc, s_id, 8)
        pacc_vec = plsc.bitcast(pbc[pl.ds(128, _LANES)], jnp.float32)
        b3, ca3 = _find_bin(hc, 8, k3)
        hi_bits = jnp.bitwise_or(lax.shift_left(b1, 19), lax.shift_left(b2, 7))
        sa3 = _level3_value_sum(hc, b3, hi_bits)
        sum12 = jnp.sum(sacc[...]) + jnp.sum(pacc_vec)

        cnt_gt = ca1 + ca2 + ca3
        vk_bits = jnp.bitwise_or(hi_bits, b3)
        vk = plsc.bitcast(jnp.full((_LANES,), vk_bits, jnp.int32), jnp.float32)
        contrib = sum12 + sa3 + (_K - cnt_gt).astype(jnp.float32) * vk
        cbuf[...] = contrib

        @pl.when(jnp.bitwise_and(vr, 1) == 0)
        def _():
            row = lax.shift_right_logical(vr, 1)
            pltpu.sync_copy(cbuf, out_hbm.at[row])

    return sc_topk


_sc_topk_call = _make_sc_topk()


@jax.jit
def _full(output, target):
    o3 = output.reshape(2 * _B, _NROW, 512)
    t3 = target.reshape(2 * _B, _NROW, 512)
    bits = _bce_bits(o3, t3)
    rows = _sc_topk_call(bits)
    return jnp.sum(rows[:, 0]) / jnp.float32(_B * _K)


def kernel(output, target):
    return _full(output, target)
